# baseline (speedup 1.0000x reference)
import functools

import numpy as np

import jax
import jax.numpy as jnp
from jax import lax
from jax.experimental import pallas as pl
from jax.experimental.pallas import tpu as pltpu
from jax.experimental.pallas import tpu_sc as plsc

N = 10000
DIN = 128
DH = 128
DOUT = 64
E = 320000

NC, NS = 2, 16
NW = NC * NS
CHUNK = 128
CH_PER_W = 80
E_PAD = NW * CH_PER_W * CHUNK
N_ACC = 10112
ROWS_PER_TILE = N_ACC // NS


def _make_sc_agg(D):
    mesh = plsc.VectorSubcoreMesh(
        core_axis_name="c", subcore_axis_name="s", num_cores=NC, num_subcores=NS
    )

    @functools.partial(
        pl.kernel,
        out_type=jax.ShapeDtypeStruct((NC, N_ACC, D), jnp.float32),
        mesh=mesh,
        scratch_types=[
            pltpu.VMEM((CH_PER_W // 2, CHUNK), jnp.int32),
            pltpu.VMEM((CH_PER_W // 2, CHUNK), jnp.int32),
            pltpu.VMEM((CHUNK, D), jnp.float32),
            pltpu.VMEM((CHUNK, D), jnp.float32),
            pltpu.VMEM_SHARED((N_ACC, D), jnp.float32),
            pltpu.SemaphoreType.DMA,
            pltpu.SemaphoreType.DMA,
            pltpu.SemaphoreType.DMA,
            pltpu.SemaphoreType.DMA,
        ],
    )
    def agg(table, src_hbm, dst_hbm, out, src_v, dst_v, buf, buf1, acc,
            sem, sem1, ssem, ssem1):
        c = lax.axis_index("c")
        s = lax.axis_index("s")
        wid = c * NS + s

        zero = jnp.zeros((16,), jnp.float32)

        @pl.loop(0, CHUNK)
        def _zero_buf(r):
            for k in range(D // 16):
                buf[r, pl.ds(k * 16, 16)] = zero

        row0 = s * ROWS_PER_TILE
        for i in range(0, ROWS_PER_TILE, CHUNK):
            rows = min(CHUNK, ROWS_PER_TILE - i)
            pltpu.sync_copy(buf.at[pl.ds(0, rows)], acc.at[pl.ds(row0 + i, rows)])
        plsc.subcore_barrier()

        def _gather(j, b, sm):
            pltpu.async_copy(table.at[src_v.at[j]], b, sm)

        def _wait(j, b, sm):
            pltpu.make_async_copy(table.at[src_v.at[j]], b, sm).wait()

        half = CH_PER_W // 2
        for hphase in range(2):
            pltpu.sync_copy(src_hbm.at[wid, pl.ds(hphase * half, half)], src_v)
            pltpu.sync_copy(dst_hbm.at[wid, pl.ds(hphase * half, half)], dst_v)
            _gather(0, buf, sem)
            _gather(1, buf1, sem1)

            @pl.loop(0, half, step=2)
            def _edges(j):
                _wait(j, buf, sem)
                pltpu.async_copy(buf, acc.at[dst_v.at[j]], ssem, add=True)
                _wait(j + 1, buf1, sem1)
                pltpu.async_copy(buf1, acc.at[dst_v.at[j + 1]], ssem1, add=True)

                pltpu.make_async_copy(buf, acc.at[dst_v.at[j]], ssem).wait()

                @pl.when(j + 2 < half)
                def _():
                    _gather(j + 2, buf, sem)

                pltpu.make_async_copy(buf1, acc.at[dst_v.at[j + 1]], ssem1).wait()

                @pl.when(j + 3 < half)
                def _():
                    _gather(j + 3, buf1, sem1)

        plsc.subcore_barrier()
        pltpu.sync_copy(acc.at[pl.ds(row0, ROWS_PER_TILE)],
                        out.at[c, pl.ds(row0, ROWS_PER_TILE)])

    return agg


_sc_agg = _make_sc_agg(DIN)


def _tc_mid_body(pa, pb, w1, b1, o):
    h = pa[0] + pb[0]
    h = jnp.dot(h, w1[...], preferred_element_type=jnp.float32) + b1[...]
    o[...] = jnp.maximum(h, 0.0)


def _tc_out_body(pa, pb, w2, b2, o):
    z = pa[0] + pb[0]
    z = jnp.dot(z, w2[...], preferred_element_type=jnp.float32) + b2[...]
    m = jnp.max(z, axis=1, keepdims=True)
    e = jnp.exp(z - m)
    lse = jnp.log(jnp.sum(e, axis=1, keepdims=True))
    o[...] = z - m - lse


_BLK = 2000

_PAD_IDS = np.arange(E_PAD - E)
_PAD_SRC = jnp.asarray(_PAD_IDS % N, dtype=jnp.int32)
_PAD_DST = jnp.asarray(N + _PAD_IDS % (N_ACC - N), dtype=jnp.int32)


def kernel(x, edge_index, W1, b1, W2, b2):
    src = edge_index[0]
    dst = edge_index[1]
    src_p = jnp.concatenate([src, _PAD_SRC])
    dst_p = jnp.concatenate([dst, _PAD_DST])
    src3 = src_p.reshape(NW, CH_PER_W, CHUNK)
    dst3 = dst_p.reshape(NW, CH_PER_W, CHUNK)

    p1 = _sc_agg(x, src3, dst3)

    h2 = pl.pallas_call(
        _tc_mid_body,
        grid=(N // _BLK,),
        in_specs=[
            pl.BlockSpec((1, _BLK, DIN), lambda i: (0, i, 0)),
            pl.BlockSpec((1, _BLK, DIN), lambda i: (1, i, 0)),
            pl.BlockSpec((DIN, DH), lambda i: (0, 0)),
            pl.BlockSpec((1, DH), lambda i: (0, 0)),
        ],
        out_specs=pl.BlockSpec((_BLK, DH), lambda i: (i, 0)),
        out_shape=jax.ShapeDtypeStruct((N, DH), jnp.float32),
    )(p1, p1, W1, b1.reshape(1, DH))

    p2 = _sc_agg(h2, src3, dst3)

    out = pl.pallas_call(
        _tc_out_body,
        grid=(N // _BLK,),
        in_specs=[
            pl.BlockSpec((1, _BLK, DH), lambda i: (0, i, 0)),
            pl.BlockSpec((1, _BLK, DH), lambda i: (1, i, 0)),
            pl.BlockSpec((DH, DOUT), lambda i: (0, 0)),
            pl.BlockSpec((1, DOUT), lambda i: (0, 0)),
        ],
        out_specs=pl.BlockSpec((_BLK, DOUT), lambda i: (i, 0)),
        out_shape=jax.ShapeDtypeStruct((N, DOUT), jnp.float32),
    )(p2, p2, W2, b2.reshape(1, DOUT))
    return out


# speedup vs baseline: 1.2766x; 1.2766x over previous
import functools

import numpy as np

import jax
import jax.numpy as jnp
from jax import lax
from jax.experimental import pallas as pl
from jax.experimental.pallas import tpu as pltpu
from jax.experimental.pallas import tpu_sc as plsc

N = 10000
DIN = 128
DH = 128
DOUT = 64
E = 320000

NC, NS = 2, 16
NW = NC * NS
CHUNK = 128
CH_PER_W = 80
E_PAD = NW * CH_PER_W * CHUNK
N_ACC = 10112
ROWS_PER_TILE = N_ACC // NS


def _make_sc_agg(D):
    mesh = plsc.VectorSubcoreMesh(
        core_axis_name="c", subcore_axis_name="s", num_cores=NC, num_subcores=NS
    )

    @functools.partial(
        pl.kernel,
        out_type=jax.ShapeDtypeStruct((NC, N_ACC, D), jnp.float32),
        mesh=mesh,
        scratch_types=[
            pltpu.VMEM((CH_PER_W // 2, CHUNK), jnp.int32),
            pltpu.VMEM((CH_PER_W // 2, CHUNK), jnp.int32),
            pltpu.VMEM((CHUNK, D), jnp.float32),
            pltpu.VMEM((CHUNK, D), jnp.float32),
            pltpu.VMEM_SHARED((N_ACC, D), jnp.float32),
            pltpu.SemaphoreType.DMA,
            pltpu.SemaphoreType.DMA,
        ],
    )
    def agg(table, src_hbm, dst_hbm, out, src_v, dst_v, buf, buf1, acc,
            sem, sem1):
        c = lax.axis_index("c")
        s = lax.axis_index("s")
        wid = c * NS + s

        zero = jnp.zeros((16,), jnp.float32)

        @pl.loop(0, CHUNK)
        def _zero_buf(r):
            for k in range(D // 16):
                buf[r, pl.ds(k * 16, 16)] = zero

        row0 = s * ROWS_PER_TILE
        for i in range(0, ROWS_PER_TILE, CHUNK):
            rows = min(CHUNK, ROWS_PER_TILE - i)
            pltpu.sync_copy(buf.at[pl.ds(0, rows)], acc.at[pl.ds(row0 + i, rows)])
        plsc.subcore_barrier()

        def _gather(j, b, sm):
            pltpu.async_copy(table.at[src_v.at[j]], b, sm)

        def _wait(j, b, sm):
            pltpu.make_async_copy(table.at[src_v.at[j]], b, sm).wait()

        half = CH_PER_W // 2
        for hphase in range(2):
            pltpu.sync_copy(src_hbm.at[wid, pl.ds(hphase * half, half)], src_v)
            pltpu.sync_copy(dst_hbm.at[wid, pl.ds(hphase * half, half)], dst_v)
            _gather(0, buf, sem)
            _gather(1, buf1, sem1)

            @pl.loop(0, half, step=2)
            def _edges(j):
                _wait(j, buf, sem)
                pltpu.sync_copy(buf, acc.at[dst_v.at[j]], add=True)

                @pl.when(j + 2 < half)
                def _():
                    _gather(j + 2, buf, sem)

                _wait(j + 1, buf1, sem1)
                pltpu.sync_copy(buf1, acc.at[dst_v.at[j + 1]], add=True)

                @pl.when(j + 3 < half)
                def _():
                    _gather(j + 3, buf1, sem1)

        plsc.subcore_barrier()
        pltpu.sync_copy(acc.at[pl.ds(row0, ROWS_PER_TILE)],
                        out.at[c, pl.ds(row0, ROWS_PER_TILE)])

    return agg


_sc_agg = _make_sc_agg(DIN)


def _tc_mid_body(pa, pb, w1, b1, o):
    h = pa[0] + pb[0]
    h = jnp.dot(h, w1[...], preferred_element_type=jnp.float32) + b1[...]
    o[...] = jnp.maximum(h, 0.0)


def _tc_out_body(pa, pb, w2, b2, o):
    z = pa[0] + pb[0]
    z = jnp.dot(z, w2[...], preferred_element_type=jnp.float32) + b2[...]
    m = jnp.max(z, axis=1, keepdims=True)
    e = jnp.exp(z - m)
    lse = jnp.log(jnp.sum(e, axis=1, keepdims=True))
    o[...] = z - m - lse


_BLK = 2000

_PAD_IDS = np.arange(E_PAD - E)
_PAD_SRC = jnp.asarray(_PAD_IDS % N, dtype=jnp.int32)
_PAD_DST = jnp.asarray(N + _PAD_IDS % (N_ACC - N), dtype=jnp.int32)


def kernel(x, edge_index, W1, b1, W2, b2):
    src = edge_index[0]
    dst = edge_index[1]
    src_p = jnp.concatenate([src, _PAD_SRC])
    dst_p = jnp.concatenate([dst, _PAD_DST])
    src3 = src_p.reshape(NW, CH_PER_W, CHUNK)
    dst3 = dst_p.reshape(NW, CH_PER_W, CHUNK)

    p1 = _sc_agg(x, src3, dst3)

    h2 = pl.pallas_call(
        _tc_mid_body,
        grid=(N // _BLK,),
        in_specs=[
            pl.BlockSpec((1, _BLK, DIN), lambda i: (0, i, 0)),
            pl.BlockSpec((1, _BLK, DIN), lambda i: (1, i, 0)),
            pl.BlockSpec((DIN, DH), lambda i: (0, 0)),
            pl.BlockSpec((1, DH), lambda i: (0, 0)),
        ],
        out_specs=pl.BlockSpec((_BLK, DH), lambda i: (i, 0)),
        out_shape=jax.ShapeDtypeStruct((N, DH), jnp.float32),
    )(p1, p1, W1, b1.reshape(1, DH))

    p2 = _sc_agg(h2, src3, dst3)

    out = pl.pallas_call(
        _tc_out_body,
        grid=(N // _BLK,),
        in_specs=[
            pl.BlockSpec((1, _BLK, DH), lambda i: (0, i, 0)),
            pl.BlockSpec((1, _BLK, DH), lambda i: (1, i, 0)),
            pl.BlockSpec((DH, DOUT), lambda i: (0, 0)),
            pl.BlockSpec((1, DOUT), lambda i: (0, 0)),
        ],
        out_specs=pl.BlockSpec((_BLK, DOUT), lambda i: (i, 0)),
        out_shape=jax.ShapeDtypeStruct((N, DOUT), jnp.float32),
    )(p2, p2, W2, b2.reshape(1, DOUT))
    return out


# speedup vs baseline: 1.3168x; 1.0315x over previous
import functools

import numpy as np

import jax
import jax.numpy as jnp
from jax import lax
from jax.experimental import pallas as pl
from jax.experimental.pallas import tpu as pltpu
from jax.experimental.pallas import tpu_sc as plsc

N = 10000
DIN = 128
DH = 128
DOUT = 64
E = 320000

NC, NS = 2, 16
NW = NC * NS
CHUNK = 128
CH_PER_W = 80
E_PAD = NW * CH_PER_W * CHUNK
N_ACC = 10112
ROWS_PER_TILE = N_ACC // NS


def _make_sc_agg(D):
    mesh = plsc.VectorSubcoreMesh(
        core_axis_name="c", subcore_axis_name="s", num_cores=NC, num_subcores=NS
    )

    @functools.partial(
        pl.kernel,
        out_type=jax.ShapeDtypeStruct((NC, N_ACC, D), jnp.float32),
        mesh=mesh,
        scratch_types=[
            pltpu.VMEM((CH_PER_W // 2, CHUNK), jnp.int32),
            pltpu.VMEM((CH_PER_W // 2, CHUNK), jnp.int32),
            pltpu.VMEM((CHUNK, D), jnp.float32),
            pltpu.VMEM((CHUNK, D), jnp.float32),
            pltpu.VMEM_SHARED((N_ACC, D), jnp.float32),
            pltpu.SemaphoreType.DMA,
            pltpu.SemaphoreType.DMA,
        ],
    )
    def agg(table, edge3, w31, out, src_v, dst_v, buf, buf1, acc,
            sem, sem1):
        c = lax.axis_index("c")
        s = lax.axis_index("s")
        wid = c * NS + s

        zero = jnp.zeros((16,), jnp.float32)

        @pl.loop(0, CHUNK)
        def _zero_buf(r):
            for k in range(D // 16):
                buf[r, pl.ds(k * 16, 16)] = zero

        row0 = s * ROWS_PER_TILE
        for i in range(0, ROWS_PER_TILE, CHUNK):
            rows = min(CHUNK, ROWS_PER_TILE - i)
            pltpu.sync_copy(buf.at[pl.ds(0, rows)], acc.at[pl.ds(row0 + i, rows)])
        plsc.subcore_barrier()

        def _gather(j, b, sm):
            pltpu.async_copy(table.at[src_v.at[j]], b, sm)

        def _wait(j, b, sm):
            pltpu.make_async_copy(table.at[src_v.at[j]], b, sm).wait()

        half = CH_PER_W // 2
        for hphase in range(2):
            @pl.when(wid < NW - 1)
            def _():
                g0 = wid * CH_PER_W + hphase * half
                pltpu.sync_copy(edge3.at[0, pl.ds(g0, half)], src_v)
                pltpu.sync_copy(edge3.at[1, pl.ds(g0, half)], dst_v)

            @pl.when(wid == NW - 1)
            def _():
                pltpu.sync_copy(w31.at[0, pl.ds(hphase * half, half)], src_v)
                pltpu.sync_copy(w31.at[1, pl.ds(hphase * half, half)], dst_v)
            _gather(0, buf, sem)
            _gather(1, buf1, sem1)

            @pl.loop(0, half, step=2)
            def _edges(j):
                _wait(j, buf, sem)
                pltpu.sync_copy(buf, acc.at[dst_v.at[j]], add=True)

                @pl.when(j + 2 < half)
                def _():
                    _gather(j + 2, buf, sem)

                _wait(j + 1, buf1, sem1)
                pltpu.sync_copy(buf1, acc.at[dst_v.at[j + 1]], add=True)

                @pl.when(j + 3 < half)
                def _():
                    _gather(j + 3, buf1, sem1)

        plsc.subcore_barrier()
        pltpu.sync_copy(acc.at[pl.ds(row0, ROWS_PER_TILE)],
                        out.at[c, pl.ds(row0, ROWS_PER_TILE)])

    return agg


_sc_agg = _make_sc_agg(DIN)


def _tc_mid_body(pa, pb, w1, b1, o):
    h = pa[0] + pb[0]
    h = jnp.dot(h, w1[...], preferred_element_type=jnp.float32) + b1[...]
    o[...] = jnp.maximum(h, 0.0)


def _tc_out_body(pa, pb, w2, b2, o):
    z = pa[0] + pb[0]
    z = jnp.dot(z, w2[...], preferred_element_type=jnp.float32) + b2[...]
    m = jnp.max(z, axis=1, keepdims=True)
    e = jnp.exp(z - m)
    lse = jnp.log(jnp.sum(e, axis=1, keepdims=True))
    o[...] = z - m - lse


_BLK = 2000

_PAD_IDS = np.arange(E_PAD - E)
_PAD31 = jnp.asarray(
    np.stack([_PAD_IDS % N, N + _PAD_IDS % (N_ACC - N)]), dtype=jnp.int32
)


def kernel(x, edge_index, W1, b1, W2, b2):
    edge3 = edge_index.reshape(2, E // CHUNK, CHUNK)
    w31 = jnp.concatenate(
        [edge_index[:, (NW - 1) * CH_PER_W * CHUNK:], _PAD31], axis=1
    ).reshape(2, CH_PER_W, CHUNK)

    p1 = _sc_agg(x, edge3, w31)

    h2 = pl.pallas_call(
        _tc_mid_body,
        grid=(N // _BLK,),
        in_specs=[
            pl.BlockSpec((1, _BLK, DIN), lambda i: (0, i, 0)),
            pl.BlockSpec((1, _BLK, DIN), lambda i: (1, i, 0)),
            pl.BlockSpec((DIN, DH), lambda i: (0, 0)),
            pl.BlockSpec((1, DH), lambda i: (0, 0)),
        ],
        out_specs=pl.BlockSpec((_BLK, DH), lambda i: (i, 0)),
        out_shape=jax.ShapeDtypeStruct((N, DH), jnp.float32),
    )(p1, p1, W1, b1.reshape(1, DH))

    p2 = _sc_agg(h2, edge3, w31)

    out = pl.pallas_call(
        _tc_out_body,
        grid=(N // _BLK,),
        in_specs=[
            pl.BlockSpec((1, _BLK, DH), lambda i: (0, i, 0)),
            pl.BlockSpec((1, _BLK, DH), lambda i: (1, i, 0)),
            pl.BlockSpec((DH, DOUT), lambda i: (0, 0)),
            pl.BlockSpec((1, DOUT), lambda i: (0, 0)),
        ],
        out_specs=pl.BlockSpec((_BLK, DOUT), lambda i: (i, 0)),
        out_shape=jax.ShapeDtypeStruct((N, DOUT), jnp.float32),
    )(p2, p2, W2, b2.reshape(1, DOUT))
    return out
